# baseline (speedup 1.0000x reference)
import functools

import jax
import jax.numpy as jnp
from jax import lax
from jax.experimental import pallas as pl
from jax.experimental.pallas import tpu as pltpu
from jax.experimental.pallas import tpu_sc as plsc

NC = 2
NS = 16
NW = NC * NS
P = 128
NB = 512
LANES = 16


def _y_matmul(x, w2, brow):
    B, C, N = x.shape
    KO = w2.shape[1]
    nblk = pl.cdiv(N, NB)

    def body(x_ref, w_ref, b_ref, y_ref):
        xb = x_ref[...].reshape(B * C, NB)
        y_ref[...] = lax.dot_general(
            xb, w_ref[...], (((0,), (0,)), ((), ())),
            preferred_element_type=jnp.float32) + b_ref[...]

    return pl.pallas_call(
        body,
        grid=(nblk,),
        in_specs=[
            pl.BlockSpec((B, C, NB), lambda i: (0, 0, i)),
            pl.BlockSpec((B * C, KO), lambda i: (0, 0)),
            pl.BlockSpec((1, KO), lambda i: (0, 0)),
        ],
        out_specs=pl.BlockSpec((NB, KO), lambda i: (i, 0)),
        out_shape=jax.ShapeDtypeStruct((N, KO), jnp.float32),
    )(x, w2, brow)


def _sc_gather_sum(y2, idx3, K, D, npad):
    T = npad // P
    G = T // NW

    mesh = plsc.VectorSubcoreMesh(
        core_axis_name="c", subcore_axis_name="s",
        num_cores=NC, num_subcores=NS)

    @functools.partial(
        pl.kernel,
        out_type=jax.ShapeDtypeStruct((npad, D), jnp.float32),
        mesh=mesh,
        scratch_types=[
            pltpu.VMEM((K, P), jnp.int32),
            pltpu.VMEM((P, D), jnp.float32),
            pltpu.SemaphoreType.DMA,
        ],
    )
    def k(y2_hbm, idx_hbm, out_hbm, idx_v, acc_v, sem):
        cid = lax.axis_index("c")
        sid = lax.axis_index("s")
        wid = sid * NC + cid
        zero = jnp.zeros((LANES,), jnp.float32)

        def chunk(g, carry):
            t = g * NW + wid
            base = t * P
            pltpu.sync_copy(idx_hbm.at[t], idx_v)

            def zero_row(p, c2):
                for cc in range(D // LANES):
                    acc_v[p, pl.ds(cc * LANES, LANES)] = zero
                return c2

            lax.fori_loop(0, P, zero_row, 0)
            handles = [
                pltpu.async_copy(y2_hbm.at[idx_v.at[kk]], acc_v, sem, add=True)
                for kk in range(K)
            ]
            for h in handles:
                h.wait()
            pltpu.sync_copy(acc_v, out_hbm.at[pl.ds(base, P)])
            return carry

        lax.fori_loop(0, G, chunk, 0)

    return k(y2, idx3)


def kernel(x, neigh_orders, W, b):
    B, C, N = x.shape
    K = neigh_orders.shape[1]
    OUT = W.shape[0]
    D = B * OUT

    wr = W.reshape(OUT, K, C).transpose(2, 1, 0)
    eyeb = jnp.eye(B, dtype=W.dtype)
    w2 = (wr[None, :, :, None, :] * eyeb[:, None, None, :, None]
          ).reshape(B * C, K * D)
    brow = jnp.tile(jnp.tile(b, B) / K, K)[None, :]

    y2 = _y_matmul(x, w2, brow).reshape(N * K, D)

    npad = ((N + NW * P - 1) // (NW * P)) * (NW * P)
    idx = neigh_orders.astype(jnp.int32) * K + jnp.arange(K, dtype=jnp.int32)[None, :]
    idx = jnp.pad(idx, ((0, npad - N), (0, 0)))
    idx3 = idx.reshape(npad // P, P, K).transpose(0, 2, 1)

    out = _sc_gather_sum(y2, idx3, K, D, npad)

    out = out[:N].reshape(N, B, OUT)
    return jnp.transpose(out, (1, 2, 0))


# speedup vs baseline: 4.2497x; 4.2497x over previous
import functools

import jax
import jax.numpy as jnp
from jax import lax
from jax.experimental import pallas as pl
from jax.experimental.pallas import tpu as pltpu
from jax.experimental.pallas import tpu_sc as plsc

NC = 2
NS = 16
NW = NC * NS
P = 128
GW = 6
NB = 512
LANES = 16


def _y_matmul(x, w2, brow):
    B, C, N = x.shape
    KO = w2.shape[1]
    nblk = pl.cdiv(N, NB)

    def body(x_ref, w_ref, b_ref, y_ref):
        xb = x_ref[...].reshape(B * C, NB)
        y_ref[...] = lax.dot_general(
            xb, w_ref[...], (((0,), (0,)), ((), ())),
            preferred_element_type=jnp.float32) + b_ref[...]

    return pl.pallas_call(
        body,
        grid=(nblk,),
        in_specs=[
            pl.BlockSpec((B, C, NB), lambda i: (0, 0, i)),
            pl.BlockSpec((B * C, KO), lambda i: (0, 0)),
            pl.BlockSpec((1, KO), lambda i: (0, 0)),
        ],
        out_specs=pl.BlockSpec((NB, KO), lambda i: (i, 0)),
        out_shape=jax.ShapeDtypeStruct((N, KO), jnp.float32),
    )(x, w2, brow)


def _sc_gather_sum(y2, idx3, idxt, K, D, G, P2, npad):
    groups = []
    left = K
    while left > 0:
        groups.append(min(GW, left))
        left -= min(GW, left)

    mesh = plsc.VectorSubcoreMesh(
        core_axis_name="c", subcore_axis_name="s",
        num_cores=NC, num_subcores=NS)

    @functools.partial(
        pl.kernel,
        out_type=jax.ShapeDtypeStruct((npad, D), jnp.float32),
        mesh=mesh,
        scratch_types=[
            pltpu.VMEM((K, P), jnp.int32),
            pltpu.VMEM((GW, P, D), jnp.float32),
            pltpu.VMEM((P, D), jnp.float32),
            pltpu.SemaphoreType.DMA,
        ],
    )
    def k(y2_hbm, idx_hbm, idxt_hbm, out_hbm, idx_v, bufs, acc_v, sem):
        cid = lax.axis_index("c")
        sid = lax.axis_index("s")
        wid = sid * NC + cid

        def do_chunk(pcur):
            def run(base):
                koff = 0
                for gi, gsz in enumerate(groups):
                    handles = [
                        pltpu.async_copy(
                            y2_hbm.at[idx_v.at[koff + j, pl.ds(0, pcur)]],
                            bufs.at[j, pl.ds(0, pcur)], sem)
                        for j in range(gsz)
                    ]
                    for h in handles:
                        h.wait()

                    def accum(p, c2):
                        for cc in range(D // LANES):
                            sl = pl.ds(cc * LANES, LANES)
                            s = bufs[0, p, sl]
                            for j in range(1, gsz):
                                s = s + bufs[j, p, sl]
                            if gi > 0:
                                s = s + acc_v[p, sl]
                            acc_v[p, sl] = s
                        return c2

                    lax.fori_loop(0, pcur, accum, 0)
                    koff += gsz
                pltpu.sync_copy(acc_v.at[pl.ds(0, pcur)],
                                out_hbm.at[pl.ds(base, pcur)])
            return run

        main_chunk = do_chunk(P)

        def chunk(g, carry):
            t = g * NW + wid
            pltpu.sync_copy(idx_hbm.at[t], idx_v)
            main_chunk(t * P)
            return carry

        lax.fori_loop(0, G, chunk, 0)

        if P2 > 0:
            tail_chunk = do_chunk(P2)

            @pl.when(wid == 0)
            def _():
                pltpu.sync_copy(idxt_hbm, idx_v)
                tail_chunk(G * NW * P)

    return k(y2, idx3, idxt)


def kernel(x, neigh_orders, W, b):
    B, C, N = x.shape
    K = neigh_orders.shape[1]
    OUT = W.shape[0]
    D = B * OUT

    wr = W.reshape(OUT, K, C).transpose(2, 1, 0)
    eyeb = jnp.eye(B, dtype=W.dtype)
    w2 = (wr[None, :, :, None, :] * eyeb[:, None, None, :, None]
          ).reshape(B * C, K * D)
    brow = jnp.tile(jnp.tile(b, B) / K, K)[None, :]

    y2 = _y_matmul(x, w2, brow).reshape(N * K, D)

    idx = neigh_orders.astype(jnp.int32) * K + jnp.arange(K, dtype=jnp.int32)[None, :]

    G = N // (NW * P)
    nmain = G * NW * P
    rem = N - nmain
    P2 = -(-rem // 8) * 8 if rem else 0
    npad = nmain + P2

    idx3 = idx[:nmain].reshape(nmain // P, P, K).transpose(0, 2, 1)
    idxt = jnp.pad(idx[nmain:], ((0, P - rem), (0, 0))).T

    out = _sc_gather_sum(y2, idx3, idxt, K, D, G, P2, npad)

    out = out[:N].reshape(N, B, OUT)
    return jnp.transpose(out, (1, 2, 0))
